# baseline (speedup 1.0000x reference)
import functools

import jax
import jax.numpy as jnp
from jax import lax
from jax.experimental import pallas as pl
from jax.experimental.pallas import tpu as pltpu

_N_CLS = 100
_CTX_LEN = 77
_N_CTX = 4
_PREFIX = 4
_EMBED = 512
_ZEROS = _CTX_LEN - _PREFIX - _N_CTX - 1
_WAVE = 10


def _tc_full(table, ctx, idx8):
    def body(idx_ref, table_ref, ctx_ref, out_ref, prompt, gsem, osem):
        for i in range(_PREFIX):
            pltpu.make_async_copy(
                table_ref.at[pl.ds(idx_ref[i], 1)], prompt.at[pl.ds(i, 1)], gsem
            ).start()
        pltpu.make_async_copy(
            table_ref.at[pl.ds(idx_ref[_PREFIX], 1)],
            prompt.at[pl.ds(_CTX_LEN - 1, 1)],
            gsem,
        ).start()
        prompt[pl.ds(_PREFIX, _N_CTX), :] = ctx_ref[...]
        prompt[pl.ds(_PREFIX + _N_CTX, _ZEROS), :] = jnp.zeros(
            (_ZEROS, _EMBED), jnp.float32
        )
        for i in range(_PREFIX):
            pltpu.make_async_copy(
                table_ref.at[pl.ds(idx_ref[i], 1)], prompt.at[pl.ds(i, 1)], gsem
            ).wait()
        pltpu.make_async_copy(
            table_ref.at[pl.ds(idx_ref[_PREFIX], 1)],
            prompt.at[pl.ds(_CTX_LEN - 1, 1)],
            gsem,
        ).wait()
        pltpu.make_async_copy(prompt, out_ref.at[0], osem.at[0]).start()
        pltpu.make_async_copy(prompt, out_ref.at[0], osem.at[0]).wait()
        done = 1
        while done < _N_CLS:
            n = min(done, _N_CLS - done)
            cp = pltpu.make_async_copy(
                out_ref.at[pl.ds(0, n)], out_ref.at[pl.ds(done, n)], osem.at[0]
            )
            cp.start()
            cp.wait()
            done += n

    grid_spec = pltpu.PrefetchScalarGridSpec(
        num_scalar_prefetch=1,
        grid=(1,),
        in_specs=[
            pl.BlockSpec(memory_space=pl.ANY),
            pl.BlockSpec(memory_space=pltpu.VMEM),
        ],
        out_specs=pl.BlockSpec(memory_space=pl.ANY),
        scratch_shapes=[
            pltpu.VMEM((_CTX_LEN, _EMBED), jnp.float32),
            pltpu.SemaphoreType.DMA,
            pltpu.SemaphoreType.DMA((_WAVE,)),
        ],
    )
    return pl.pallas_call(
        body,
        grid_spec=grid_spec,
        out_shape=jax.ShapeDtypeStruct((_N_CLS, _CTX_LEN, _EMBED), jnp.float32),
    )(idx8, table, ctx)


def kernel(token_embedding, ctx_vectors, tokenized_prompt):
    idx8 = jnp.concatenate(
        [
            tokenized_prompt[:_PREFIX],
            tokenized_prompt[_CTX_LEN - 1 :],
            jnp.zeros((3,), jnp.int32),
        ]
    )
    return _tc_full(token_embedding, ctx_vectors, idx8)


# speedup vs baseline: 40.5287x; 40.5287x over previous
import jax
import jax.numpy as jnp
from jax.experimental import pallas as pl
from jax.experimental.pallas import tpu as pltpu

_N_CLS = 100
_CTX_LEN = 77
_N_CTX = 4
_PREFIX = 4
_EMBED = 512
_ZEROS = _CTX_LEN - _PREFIX - _N_CTX - 1
_CB = 10


def _gather_copies(idx_ref, table_ref, prompt, gsem):
    copies = [
        pltpu.make_async_copy(
            table_ref.at[pl.ds(idx_ref[i], 1)], prompt.at[pl.ds(i, 1)], gsem
        )
        for i in range(_PREFIX)
    ]
    copies.append(
        pltpu.make_async_copy(
            table_ref.at[pl.ds(idx_ref[_PREFIX], 1)],
            prompt.at[pl.ds(_CTX_LEN - 1, 1)],
            gsem,
        )
    )
    return copies


def _tc_full(table, ctx, idx8):
    def body(idx_ref, table_ref, ctx_ref, o_ref, prompt, gsem):
        @pl.when(pl.program_id(0) == 0)
        def _():
            for cp in _gather_copies(idx_ref, table_ref, prompt, gsem):
                cp.start()
            prompt[pl.ds(_PREFIX, _N_CTX), :] = ctx_ref[...]
            prompt[pl.ds(_PREFIX + _N_CTX, _ZEROS), :] = jnp.zeros(
                (_ZEROS, _EMBED), jnp.float32
            )
            for cp in _gather_copies(idx_ref, table_ref, prompt, gsem):
                cp.wait()

        o_ref[...] = jnp.broadcast_to(
            prompt[...][None], (_CB, _CTX_LEN, _EMBED)
        )

    grid_spec = pltpu.PrefetchScalarGridSpec(
        num_scalar_prefetch=1,
        grid=(_N_CLS // _CB,),
        in_specs=[
            pl.BlockSpec(memory_space=pl.ANY),
            pl.BlockSpec((_N_CTX, _EMBED), lambda i, idx: (0, 0)),
        ],
        out_specs=pl.BlockSpec((_CB, _CTX_LEN, _EMBED), lambda i, idx: (i, 0, 0)),
        scratch_shapes=[
            pltpu.VMEM((_CTX_LEN, _EMBED), jnp.float32),
            pltpu.SemaphoreType.DMA,
        ],
    )
    return pl.pallas_call(
        body,
        grid_spec=grid_spec,
        out_shape=jax.ShapeDtypeStruct((_N_CLS, _CTX_LEN, _EMBED), jnp.float32),
    )(idx8, table, ctx)


def kernel(token_embedding, ctx_vectors, tokenized_prompt):
    idx8 = jnp.concatenate(
        [
            tokenized_prompt[:_PREFIX],
            tokenized_prompt[_CTX_LEN - 1 :],
            jnp.zeros((3,), jnp.int32),
        ]
    )
    return _tc_full(token_embedding, ctx_vectors, idx8)


# speedup vs baseline: 43.6436x; 1.0769x over previous
import jax
import jax.numpy as jnp
from jax.experimental import pallas as pl
from jax.experimental.pallas import tpu as pltpu

_N_CLS = 100
_CTX_LEN = 77
_N_CTX = 4
_PREFIX = 4
_EMBED = 512
_ZEROS = _CTX_LEN - _PREFIX - _N_CTX - 1
_CB = 20


def _gather_copies(idx_ref, table_ref, prompt, gsem):
    copies = [
        pltpu.make_async_copy(
            table_ref.at[pl.ds(idx_ref[i], 1)], prompt.at[pl.ds(i, 1)], gsem
        )
        for i in range(_PREFIX)
    ]
    copies.append(
        pltpu.make_async_copy(
            table_ref.at[pl.ds(idx_ref[_PREFIX], 1)],
            prompt.at[pl.ds(_CTX_LEN - 1, 1)],
            gsem,
        )
    )
    return copies


def _tc_full(table, ctx, idx8):
    def body(idx_ref, table_ref, ctx_ref, o_ref, prompt, gsem):
        @pl.when(pl.program_id(0) == 0)
        def _():
            for cp in _gather_copies(idx_ref, table_ref, prompt, gsem):
                cp.start()
            prompt[pl.ds(_PREFIX, _N_CTX), :] = ctx_ref[...]
            prompt[pl.ds(_PREFIX + _N_CTX, _ZEROS), :] = jnp.zeros(
                (_ZEROS, _EMBED), jnp.float32
            )
            for cp in _gather_copies(idx_ref, table_ref, prompt, gsem):
                cp.wait()

        o_ref[...] = jnp.broadcast_to(
            prompt[...][None], (_CB, _CTX_LEN, _EMBED)
        )

    grid_spec = pltpu.PrefetchScalarGridSpec(
        num_scalar_prefetch=1,
        grid=(_N_CLS // _CB,),
        in_specs=[
            pl.BlockSpec(memory_space=pl.ANY),
            pl.BlockSpec((_N_CTX, _EMBED), lambda i, idx: (0, 0)),
        ],
        out_specs=pl.BlockSpec((_CB, _CTX_LEN, _EMBED), lambda i, idx: (i, 0, 0)),
        scratch_shapes=[
            pltpu.VMEM((_CTX_LEN, _EMBED), jnp.float32),
            pltpu.SemaphoreType.DMA,
        ],
    )
    return pl.pallas_call(
        body,
        grid_spec=grid_spec,
        out_shape=jax.ShapeDtypeStruct((_N_CLS, _CTX_LEN, _EMBED), jnp.float32),
    )(idx8, table, ctx)


def kernel(token_embedding, ctx_vectors, tokenized_prompt):
    idx8 = jnp.concatenate(
        [
            tokenized_prompt[:_PREFIX],
            tokenized_prompt[_CTX_LEN - 1 :],
            jnp.zeros((3,), jnp.int32),
        ]
    )
    return _tc_full(token_embedding, ctx_vectors, idx8)
